# baseline (speedup 1.0000x reference)
import dataclasses
import functools

import jax
import jax.numpy as jnp
from jax import lax
from jax.experimental import pallas as pl
from jax.experimental.pallas import tpu as pltpu
from jax.experimental.pallas import tpu_sc as plsc

_B = 16384
_D = 64
_NC = 2
_NS = 16
_L = 16
_NW = _NC * _NS
_BPW = _B // _NW
_CHUNK = 128
_NCHUNK = _BPW // _CHUNK


def _compiler_params():
    cp = pltpu.CompilerParams(use_tc_tiling_on_sc=False)
    if "needs_layout_passes" in pltpu.CompilerParams.__dataclass_fields__:
        cp = dataclasses.replace(cp, needs_layout_passes=False)
    return cp


def _make_kernel():
    mesh = plsc.VectorSubcoreMesh(
        core_axis_name="c", subcore_axis_name="s",
        num_cores=_NC, num_subcores=_NS,
    )

    @functools.partial(
        pl.kernel,
        out_type=jax.ShapeDtypeStruct((_NW, _L), jnp.float32),
        mesh=mesh,
        scratch_types=[
            pltpu.VMEM((_NCHUNK, _CHUNK), jnp.int32),
            pltpu.VMEM((_BPW, _D), jnp.float32),
            pltpu.VMEM((_BPW, _D), jnp.float32),
            pltpu.VMEM((_L,), jnp.float32),
            pltpu.SemaphoreType.DMA,
            pltpu.SemaphoreType.DMA,
        ],
        compiler_params=_compiler_params(),
    )
    def center_loss_kernel(x_hbm, lab_hbm, ctr_hbm, out_hbm,
                           idx_v, x_v, c_v, acc_v, gsem, xsem):
        wid = lax.axis_index("s") * _NC + lax.axis_index("c")
        base = wid * _BPW

        pltpu.sync_copy(lab_hbm.at[wid], idx_v)

        xcp = pltpu.async_copy(x_hbm.at[pl.ds(base, _BPW)], x_v, xsem)
        gcps = []
        for k in range(_NCHUNK):
            gcps.append(
                pltpu.async_copy(
                    ctr_hbm.at[idx_v.at[k]],
                    c_v.at[pl.ds(k * _CHUNK, _CHUNK)],
                    gsem,
                )
            )
        xcp.wait()
        for cp in gcps:
            cp.wait()

        acc_v[...] = jnp.zeros((_L,), jnp.float32)
        lanes = lax.iota(jnp.int32, _L)

        @pl.loop(0, _BPW, step=_L)
        def _(r):
            rows = lanes + r
            parts = [jnp.zeros((_L,), jnp.float32) for _ in range(4)]
            for f in range(_D):
                col = jnp.full((_L,), f, jnp.int32)
                xi = plsc.load_gather(x_v, [rows, col])
                ci = plsc.load_gather(c_v, [rows, col])
                d = xi - ci
                parts[f % 4] = parts[f % 4] + d * d
            dist = (parts[0] + parts[1]) + (parts[2] + parts[3])
            dist = jnp.minimum(jnp.maximum(dist, 1e-12), 1e12)
            acc_v[...] = acc_v[...] + dist

        pltpu.sync_copy(acc_v, out_hbm.at[wid])

    return center_loss_kernel


_KERNEL = _make_kernel()


def kernel(x, labels, centers):
    lab = labels.astype(jnp.int32).reshape(_NW, _NCHUNK, _CHUNK)
    partials = _KERNEL(x, lab, centers)
    return jnp.sum(partials) * (1.0 / _B)


# speedup vs baseline: 2.1590x; 2.1590x over previous
import dataclasses
import functools

import jax
import jax.numpy as jnp
from jax import lax
from jax.experimental import pallas as pl
from jax.experimental.pallas import tpu as pltpu
from jax.experimental.pallas import tpu_sc as plsc

_B = 16384
_D = 64
_V = 100000
_NC = 2
_NS = 16
_L = 16
_NW = _NC * _NS
_FPW = _D // _NW
_BCHUNK = 4096
_NBCH = _B // _BCHUNK


def _sc_compiler_params():
    cp = pltpu.CompilerParams()
    if "needs_layout_passes" in pltpu.CompilerParams.__dataclass_fields__:
        cp = dataclasses.replace(cp, needs_layout_passes=False)
    return cp


def _make_gather_kernel():
    mesh = plsc.VectorSubcoreMesh(
        core_axis_name="c", subcore_axis_name="s",
        num_cores=_NC, num_subcores=_NS,
    )

    @functools.partial(
        pl.kernel,
        out_type=jax.ShapeDtypeStruct((_D, _B), jnp.float32),
        mesh=mesh,
        scratch_types=[
            pltpu.VMEM((_V,), jnp.float32),
            pltpu.VMEM((_BCHUNK,), jnp.int32),
            pltpu.VMEM((_BCHUNK,), jnp.float32),
            pltpu.SemaphoreType.DMA,
        ],
        compiler_params=_sc_compiler_params(),
    )
    def gather_kernel(ct_hbm, lab_hbm, out_hbm, row_v, lab_v, o_v, sem):
        wid = lax.axis_index("s") * _NC + lax.axis_index("c")
        for t in range(_FPW):
            f = wid * _FPW + t
            pltpu.async_copy(ct_hbm.at[f], row_v, sem).wait()
            for k in range(_NBCH):
                pltpu.sync_copy(lab_hbm.at[pl.ds(k * _BCHUNK, _BCHUNK)], lab_v)

                @pl.loop(0, _BCHUNK, step=_L)
                def _(j):
                    idx = lab_v[pl.ds(j, _L)]
                    o_v[pl.ds(j, _L)] = plsc.load_gather(row_v, [idx])

                pltpu.sync_copy(o_v, out_hbm.at[f, pl.ds(k * _BCHUNK, _BCHUNK)])

    return gather_kernel


_GATHER = _make_gather_kernel()

_BC = 2048


def _loss_body(xt_ref, g_ref, o_ref):
    i = pl.program_id(0)
    d = xt_ref[...] - g_ref[...]
    s = jnp.sum(d * d, axis=0, keepdims=True)
    s = jnp.minimum(jnp.maximum(s, 1e-12), 1e12)
    part = jnp.sum(s) * (1.0 / _B)

    @pl.when(i == 0)
    def _():
        o_ref[...] = jnp.zeros((1, 128), jnp.float32)

    o_ref[...] += jnp.full((1, 128), part, jnp.float32)


_LOSS = pl.pallas_call(
    _loss_body,
    out_shape=jax.ShapeDtypeStruct((1, 128), jnp.float32),
    grid=(_B // _BC,),
    in_specs=[
        pl.BlockSpec((_D, _BC), lambda i: (0, i)),
        pl.BlockSpec((_D, _BC), lambda i: (0, i)),
    ],
    out_specs=pl.BlockSpec((1, 128), lambda i: (0, 0)),
)


def kernel(x, labels, centers):
    xt = x.T
    ct = centers.T
    lab = labels.astype(jnp.int32)
    g = _GATHER(ct, lab)
    return _LOSS(xt, g)[0, 0]


# speedup vs baseline: 2.3483x; 1.0877x over previous
import dataclasses
import functools

import jax
import jax.numpy as jnp
from jax import lax
from jax.experimental import pallas as pl
from jax.experimental.pallas import tpu as pltpu
from jax.experimental.pallas import tpu_sc as plsc

_B = 16384
_D = 64
_V = 100000
_NC = 2
_NS = 16
_L = 16
_NW = _NC * _NS
_FPW = _D // _NW
_BCHUNK = 4096
_NBCH = _B // _BCHUNK


def _sc_compiler_params():
    cp = pltpu.CompilerParams()
    if "needs_layout_passes" in pltpu.CompilerParams.__dataclass_fields__:
        cp = dataclasses.replace(cp, needs_layout_passes=False)
    return cp


def _make_gather_kernel():
    mesh = plsc.VectorSubcoreMesh(
        core_axis_name="c", subcore_axis_name="s",
        num_cores=_NC, num_subcores=_NS,
    )

    @functools.partial(
        pl.kernel,
        out_type=jax.ShapeDtypeStruct((_D, _B), jnp.float32),
        mesh=mesh,
        scratch_types=[
            pltpu.VMEM((_V,), jnp.float32),
            pltpu.VMEM((_B,), jnp.int32),
            pltpu.VMEM((_BCHUNK,), jnp.float32),
            pltpu.VMEM((_BCHUNK,), jnp.float32),
            pltpu.SemaphoreType.DMA,
            pltpu.SemaphoreType.DMA,
            pltpu.SemaphoreType.DMA,
        ],
        compiler_params=_sc_compiler_params(),
    )
    def gather_kernel(ct_hbm, lab_hbm, out_hbm, row_v, lab_v, o0_v, o1_v,
                      rsem, lsem, osem):
        o_bufs = (o0_v, o1_v)
        wid = lax.axis_index("s") * _NC + lax.axis_index("c")
        lcp = pltpu.async_copy(lab_hbm, lab_v, lsem)
        rcp = pltpu.async_copy(ct_hbm.at[wid * _FPW], row_v, rsem)
        lcp.wait()
        ocps = [None, None]
        for t in range(_FPW):
            f = wid * _FPW + t
            rcp.wait()
            for k in range(_NBCH):
                buf = k % 2
                if ocps[buf] is not None:
                    ocps[buf].wait()

                @pl.loop(0, _BCHUNK, step=4 * _L)
                def _(j):
                    for u in range(4):
                        idx = lab_v[pl.ds(k * _BCHUNK + j + u * _L, _L)]
                        o_bufs[buf][pl.ds(j + u * _L, _L)] = (
                            plsc.load_gather(row_v, [idx]))

                if t + 1 == _FPW and k + 1 == _NBCH:
                    pass
                elif k + 1 == _NBCH:
                    rcp = pltpu.async_copy(ct_hbm.at[f + 1], row_v, rsem)
                ocps[buf] = pltpu.async_copy(
                    o_bufs[buf],
                    out_hbm.at[f, pl.ds(k * _BCHUNK, _BCHUNK)],
                    osem)
        for cp in ocps:
            if cp is not None:
                cp.wait()

    return gather_kernel


_GATHER = _make_gather_kernel()

_BC = 2048


def _loss_body(xt_ref, g_ref, o_ref):
    i = pl.program_id(0)
    d = xt_ref[...] - g_ref[...]
    s = jnp.sum(d * d, axis=0, keepdims=True)
    s = jnp.minimum(jnp.maximum(s, 1e-12), 1e12)
    part = jnp.sum(s) * (1.0 / _B)

    @pl.when(i == 0)
    def _():
        o_ref[...] = jnp.zeros((1, 128), jnp.float32)

    o_ref[...] += jnp.full((1, 128), part, jnp.float32)


_LOSS = pl.pallas_call(
    _loss_body,
    out_shape=jax.ShapeDtypeStruct((1, 128), jnp.float32),
    grid=(_B // _BC,),
    in_specs=[
        pl.BlockSpec((_D, _BC), lambda i: (0, i)),
        pl.BlockSpec((_D, _BC), lambda i: (0, i)),
    ],
    out_specs=pl.BlockSpec((1, 128), lambda i: (0, 0)),
)


def kernel(x, labels, centers):
    xt = x.T
    ct = centers.T
    lab = labels.astype(jnp.int32)
    g = _GATHER(ct, lab)
    return _LOSS(xt, g)[0, 0]


# speedup vs baseline: 3.0444x; 1.2964x over previous
import dataclasses
import functools

import jax
import jax.numpy as jnp
from jax import lax
from jax.experimental import pallas as pl
from jax.experimental.pallas import tpu as pltpu
from jax.experimental.pallas import tpu_sc as plsc

_B = 16384
_D = 64
_V = 100000
_NC = 2
_NS = 16
_L = 16
_NW = _NC * _NS
_FPW = _D // _NW
_BCHUNK = 4096
_NBCH = _B // _BCHUNK


def _sc_compiler_params():
    cp = pltpu.CompilerParams()
    if "needs_layout_passes" in pltpu.CompilerParams.__dataclass_fields__:
        cp = dataclasses.replace(cp, needs_layout_passes=False)
    return cp


def _make_gather_kernel():
    mesh = plsc.VectorSubcoreMesh(
        core_axis_name="c", subcore_axis_name="s",
        num_cores=_NC, num_subcores=_NS,
    )

    @functools.partial(
        pl.kernel,
        out_type=jax.ShapeDtypeStruct((_D, _B), jnp.float32),
        mesh=mesh,
        scratch_types=[
            pltpu.VMEM((_V,), jnp.float32),
            pltpu.VMEM((_B,), jnp.int32),
            pltpu.VMEM((_BCHUNK,), jnp.float32),
            pltpu.VMEM((_BCHUNK,), jnp.float32),
            pltpu.SemaphoreType.DMA,
            pltpu.SemaphoreType.DMA,
            pltpu.SemaphoreType.DMA,
        ],
        compiler_params=_sc_compiler_params(),
    )
    def gather_kernel(ct_hbm, lab_hbm, out_hbm, row_v, lab_v, o0_v, o1_v,
                      rsem, lsem, osem):
        o_bufs = (o0_v, o1_v)
        wid = lax.axis_index("s") * _NC + lax.axis_index("c")
        lcp = pltpu.async_copy(lab_hbm, lab_v, lsem)
        rcp = pltpu.async_copy(ct_hbm.at[wid * _FPW], row_v, rsem)
        lcp.wait()
        ocps = [None, None]
        for t in range(_FPW):
            f = wid * _FPW + t
            rcp.wait()
            for k in range(_NBCH):
                buf = k % 2
                if ocps[buf] is not None:
                    ocps[buf].wait()

                @pl.loop(0, _BCHUNK, step=8 * _L)
                def _(j):
                    idxs = [lab_v[pl.ds(k * _BCHUNK + j + u * _L, _L)]
                            for u in range(8)]
                    gs = [plsc.load_gather(row_v, [idxs[u]]) for u in range(8)]
                    for u in range(8):
                        o_bufs[buf][pl.ds(j + u * _L, _L)] = gs[u]

                if t + 1 == _FPW and k + 1 == _NBCH:
                    pass
                elif k + 1 == _NBCH:
                    rcp = pltpu.async_copy(ct_hbm.at[f + 1], row_v, rsem)
                ocps[buf] = pltpu.async_copy(
                    o_bufs[buf],
                    out_hbm.at[f, pl.ds(k * _BCHUNK, _BCHUNK)],
                    osem)
        for cp in ocps:
            if cp is not None:
                cp.wait()

    return gather_kernel


_GATHER = _make_gather_kernel()

_BC = 2048


def _loss_body(xt_ref, g_ref, o_ref):
    i = pl.program_id(0)
    d = xt_ref[...] - g_ref[...]
    s = jnp.sum(d * d, axis=0, keepdims=True)
    s = jnp.minimum(jnp.maximum(s, 1e-12), 1e12)
    part = jnp.sum(s) * (1.0 / _B)

    @pl.when(i == 0)
    def _():
        o_ref[...] = jnp.zeros((1, 128), jnp.float32)

    o_ref[...] += jnp.full((1, 128), part, jnp.float32)


_LOSS = pl.pallas_call(
    _loss_body,
    out_shape=jax.ShapeDtypeStruct((1, 128), jnp.float32),
    grid=(_B // _BC,),
    in_specs=[
        pl.BlockSpec((_D, _BC), lambda i: (0, i)),
        pl.BlockSpec((_D, _BC), lambda i: (0, i)),
    ],
    out_specs=pl.BlockSpec((1, 128), lambda i: (0, 0)),
)


def kernel(x, labels, centers):
    xt = x.T
    ct = centers.T
    lab = labels.astype(jnp.int32)
    g = _GATHER(ct, lab)
    return _LOSS(xt, g)[0, 0]
